# baseline (speedup 1.0000x reference)
import functools

import jax
import jax.numpy as jnp
from jax import lax
from jax.experimental import pallas as pl
from jax.experimental.pallas import tpu as pltpu
from jax.experimental.pallas import tpu_sc as plsc

VOCAB = 1000000
EMBED_DIM = 64
PADDED = 128
MAX_SEQ = 512
B, L = 4096, 200

N = B * L
IW = 100
PAGES = N // IW
NW = 32
STEPS = PAGES // NW
NBUF = 2


def _make_kernel():
    mesh = plsc.VectorSubcoreMesh(core_axis_name="c", subcore_axis_name="s")

    scratch = []
    for _ in range(NBUF):
        scratch += [
            pltpu.VMEM((IW,), jnp.int32),
            pltpu.VMEM((IW,), jnp.int32),
            pltpu.VMEM((IW, PADDED), jnp.float32),
            pltpu.VMEM((IW, PADDED), jnp.float32),
            pltpu.VMEM((IW, EMBED_DIM), jnp.float32),
            pltpu.SemaphoreType.DMA,
        ]

    @functools.partial(
        pl.kernel,
        mesh=mesh,
        out_type=jax.ShapeDtypeStruct((PAGES, IW, EMBED_DIM), jnp.float32),
        scratch_types=scratch,
    )
    def emb_kernel(x_hbm, pos_hbm, tok_hbm, pe_hbm, out_hbm, *bufs):
        idxt = [bufs[6 * s + 0] for s in range(NBUF)]
        idxp = [bufs[6 * s + 1] for s in range(NBUF)]
        a = [bufs[6 * s + 2] for s in range(NBUF)]
        b = [bufs[6 * s + 3] for s in range(NBUF)]
        c = [bufs[6 * s + 4] for s in range(NBUF)]
        sem = [bufs[6 * s + 5] for s in range(NBUF)]

        wid = lax.axis_index("s") * 2 + lax.axis_index("c")
        page_base = wid * STEPS

        def issue(i, s):
            p = page_base + i
            pltpu.sync_copy(x_hbm.at[p], idxt[s])
            pltpu.sync_copy(pos_hbm.at[p], idxp[s])
            pltpu.async_copy(tok_hbm.at[idxt[s]], a[s], sem[s])
            pltpu.async_copy(pe_hbm.at[idxp[s]], b[s], sem[s])

        def finish(i, s):
            pltpu.make_async_copy(tok_hbm.at[idxt[s]], a[s], sem[s]).wait()
            pltpu.make_async_copy(pe_hbm.at[idxp[s]], b[s], sem[s]).wait()

            @pl.loop(0, IW, unroll=10)
            def add_row(r):
                for col in range(EMBED_DIM // 16):
                    sl = (r, pl.ds(col * 16, 16))
                    c[s][sl] = a[s][sl] + b[s][sl]

            pltpu.sync_copy(c[s], out_hbm.at[page_base + i])

        issue(0, 0)

        def pair(m, carry):
            issue(2 * m + 1, 1)
            finish(2 * m, 0)

            @pl.when(2 * m + 2 < STEPS)
            def _():
                issue(2 * m + 2, 0)

            finish(2 * m + 1, 1)
            return carry

        lax.fori_loop(0, STEPS // 2, pair, 0)

    return emb_kernel


_emb = _make_kernel()


@jax.jit
def kernel(x, pos, token_table, position_table):
    tok_p = jnp.pad(token_table, ((0, 0), (0, PADDED - EMBED_DIM)))
    pe_p = jnp.pad(position_table, ((0, 0), (0, PADDED - EMBED_DIM)))
    out = _emb(x.reshape(PAGES, IW), pos.reshape(PAGES, IW), tok_p, pe_p)
    return out.reshape(B, L, EMBED_DIM)


# speedup vs baseline: 1.2938x; 1.2938x over previous
import functools

import jax
import jax.numpy as jnp
from jax import lax
from jax.experimental import pallas as pl
from jax.experimental.pallas import tpu as pltpu
from jax.experimental.pallas import tpu_sc as plsc

VOCAB = 1000000
EMBED_DIM = 64
PADDED = 128
MAX_SEQ = 512
B, L = 4096, 200

N = B * L
IW = 128
PAGES = N // IW
NW = 32
STEPS = PAGES // NW
NBUF = 2


def _make_kernel():
    mesh = plsc.VectorSubcoreMesh(core_axis_name="c", subcore_axis_name="s")

    scratch = []
    for _ in range(NBUF):
        scratch += [
            pltpu.VMEM((IW,), jnp.int32),
            pltpu.VMEM((IW,), jnp.int32),
            pltpu.VMEM((IW, PADDED), jnp.float32),
            pltpu.VMEM((IW, PADDED), jnp.float32),
            pltpu.SemaphoreType.DMA,
        ]
    scratch += [
        pltpu.VMEM((IW, EMBED_DIM), jnp.float32),
    ]

    @functools.partial(
        pl.kernel,
        mesh=mesh,
        out_type=jax.ShapeDtypeStruct((PAGES, IW, EMBED_DIM), jnp.float32),
        scratch_types=scratch,
    )
    def emb_kernel(x_hbm, pos_hbm, tok_hbm, pe_hbm, out_hbm, *bufs):
        idxt = [bufs[5 * s + 0] for s in range(NBUF)]
        idxp = [bufs[5 * s + 1] for s in range(NBUF)]
        a = [bufs[5 * s + 2] for s in range(NBUF)]
        b = [bufs[5 * s + 3] for s in range(NBUF)]
        sem = [bufs[5 * s + 4] for s in range(NBUF)]
        c = bufs[5 * NBUF]

        wid = lax.axis_index("s") * 2 + lax.axis_index("c")
        page_base = wid * STEPS

        def issue(i, s):
            p = page_base + i
            pltpu.sync_copy(x_hbm.at[p], idxt[s])
            pltpu.sync_copy(pos_hbm.at[p], idxp[s])
            pltpu.async_copy(tok_hbm.at[idxt[s]], a[s], sem[s])
            pltpu.async_copy(pe_hbm.at[idxp[s]], b[s], sem[s])

        def finish(i, s):
            pltpu.make_async_copy(tok_hbm.at[idxt[s]], a[s], sem[s]).wait()
            pltpu.make_async_copy(pe_hbm.at[idxp[s]], b[s], sem[s]).wait()

            @pl.loop(0, IW, unroll=10)
            def add_row(r):
                for col in range(EMBED_DIM // 16):
                    sl = (r, pl.ds(col * 16, 16))
                    c[sl] = a[s][sl] + b[s][sl]

            pltpu.sync_copy(c, out_hbm.at[page_base + i])

        issue(0, 0)

        def pair(m, carry):
            issue(2 * m + 1, 1)
            finish(2 * m, 0)

            @pl.when(2 * m + 2 < STEPS)
            def _():
                issue(2 * m + 2, 0)

            finish(2 * m + 1, 1)
            return carry

        lax.fori_loop(0, STEPS // 2, pair, 0)

    return emb_kernel


_emb = _make_kernel()


@jax.jit
def kernel(x, pos, token_table, position_table):
    tok_p = jnp.pad(token_table, ((0, 0), (0, PADDED - EMBED_DIM)))
    pe_p = jnp.pad(position_table, ((0, 0), (0, PADDED - EMBED_DIM)))
    out = _emb(x.reshape(PAGES, IW), pos.reshape(PAGES, IW), tok_p, pe_p)
    return out.reshape(B, L, EMBED_DIM)


# speedup vs baseline: 1.2969x; 1.0024x over previous
import functools

import jax
import jax.numpy as jnp
from jax import lax
from jax.experimental import pallas as pl
from jax.experimental.pallas import tpu as pltpu
from jax.experimental.pallas import tpu_sc as plsc

VOCAB = 1000000
EMBED_DIM = 64
PADDED = 128
MAX_SEQ = 512
B, L = 4096, 200

N = B * L
IW = 128
PAGES = N // IW
NW = 32
STEPS = PAGES // NW
NBUF = 2


def _make_kernel():
    mesh = plsc.VectorSubcoreMesh(core_axis_name="c", subcore_axis_name="s")

    scratch = []
    for _ in range(NBUF):
        scratch += [
            pltpu.VMEM((IW,), jnp.int32),
            pltpu.VMEM((IW,), jnp.int32),
            pltpu.VMEM((IW, PADDED), jnp.float32),
            pltpu.SemaphoreType.DMA,
        ]
    scratch += [
        pltpu.VMEM((IW, EMBED_DIM), jnp.float32),
        pltpu.VMEM((MAX_SEQ, EMBED_DIM), jnp.float32),
    ]

    @functools.partial(
        pl.kernel,
        mesh=mesh,
        out_type=jax.ShapeDtypeStruct((PAGES, IW, EMBED_DIM), jnp.float32),
        scratch_types=scratch,
    )
    def emb_kernel(x_hbm, pos_hbm, tok_hbm, pe_hbm, out_hbm, *bufs):
        idxt = [bufs[4 * s + 0] for s in range(NBUF)]
        idxp = [bufs[4 * s + 1] for s in range(NBUF)]
        a = [bufs[4 * s + 2] for s in range(NBUF)]
        sem = [bufs[4 * s + 3] for s in range(NBUF)]
        c = bufs[4 * NBUF]
        pev = bufs[4 * NBUF + 1]

        wid = lax.axis_index("s") * 2 + lax.axis_index("c")
        page_base = wid * STEPS

        pltpu.sync_copy(pe_hbm, pev)

        def issue(i, s):
            p = page_base + i
            pltpu.sync_copy(x_hbm.at[p], idxt[s])
            pltpu.sync_copy(pos_hbm.at[p], idxp[s])
            pltpu.async_copy(tok_hbm.at[idxt[s]], a[s], sem[s])

        def finish(i, s):
            pltpu.make_async_copy(tok_hbm.at[idxt[s]], a[s], sem[s]).wait()

            @pl.loop(0, IW // 16, unroll=2)
            def add_chunk(ch):
                pv = idxp[s][pl.ds(ch * 16, 16)]
                for k in range(16):
                    r = ch * 16 + k
                    p = pv[k]
                    for col in range(EMBED_DIM // 16):
                        sl = (r, pl.ds(col * 16, 16))
                        c[sl] = a[s][sl] + pev[p, pl.ds(col * 16, 16)]

            pltpu.sync_copy(c, out_hbm.at[page_base + i])

        issue(0, 0)

        def pair(m, carry):
            issue(2 * m + 1, 1)
            finish(2 * m, 0)

            @pl.when(2 * m + 2 < STEPS)
            def _():
                issue(2 * m + 2, 0)

            finish(2 * m + 1, 1)
            return carry

        lax.fori_loop(0, STEPS // 2, pair, 0)

    return emb_kernel


_emb = _make_kernel()


@jax.jit
def kernel(x, pos, token_table, position_table):
    tok_p = jnp.pad(token_table, ((0, 0), (0, PADDED - EMBED_DIM)))
    out = _emb(x.reshape(PAGES, IW), pos.reshape(PAGES, IW), tok_p,
               position_table)
    return out.reshape(B, L, EMBED_DIM)


# speedup vs baseline: 1.4002x; 1.0797x over previous
import functools

import jax
import jax.numpy as jnp
from jax import lax
from jax.experimental import pallas as pl
from jax.experimental.pallas import tpu as pltpu
from jax.experimental.pallas import tpu_sc as plsc

VOCAB = 1000000
EMBED_DIM = 64
PADDED = 128
MAX_SEQ = 512
B, L = 4096, 200

N = B * L
IW = 128
PAGES = N // IW
NW = 32
STEPS = PAGES // NW
NBUF = 2


def _make_kernel():
    mesh = plsc.VectorSubcoreMesh(core_axis_name="c", subcore_axis_name="s")

    scratch = []
    for _ in range(NBUF):
        scratch += [
            pltpu.VMEM((IW,), jnp.int32),
            pltpu.VMEM((IW,), jnp.int32),
            pltpu.VMEM((IW, PADDED), jnp.float32),
            pltpu.SemaphoreType.DMA,
        ]
    scratch += [
        pltpu.VMEM((IW, EMBED_DIM), jnp.float32),
        pltpu.VMEM((IW, EMBED_DIM), jnp.float32),
        pltpu.SemaphoreType.DMA,
        pltpu.SemaphoreType.DMA,
        pltpu.VMEM((MAX_SEQ // 2, 2 * EMBED_DIM), jnp.float32),
    ]

    @functools.partial(
        pl.kernel,
        mesh=mesh,
        out_type=jax.ShapeDtypeStruct((PAGES, IW, EMBED_DIM), jnp.float32),
        scratch_types=scratch,
    )
    def emb_kernel(x_hbm, pos_hbm, tok_hbm, pe_hbm, out_hbm, *bufs):
        idxt = [bufs[4 * s + 0] for s in range(NBUF)]
        idxp = [bufs[4 * s + 1] for s in range(NBUF)]
        a = [bufs[4 * s + 2] for s in range(NBUF)]
        sem = [bufs[4 * s + 3] for s in range(NBUF)]
        c = [bufs[4 * NBUF], bufs[4 * NBUF + 1]]
        wsem = [bufs[4 * NBUF + 2], bufs[4 * NBUF + 3]]
        pev = bufs[4 * NBUF + 4]

        wid = lax.axis_index("s") * 2 + lax.axis_index("c")
        page_base = wid * STEPS

        pltpu.sync_copy(pe_hbm, pev)

        def issue(i, s):
            p = page_base + i
            pltpu.sync_copy(x_hbm.at[p], idxt[s])
            pltpu.sync_copy(pos_hbm.at[p], idxp[s])
            pltpu.async_copy(tok_hbm.at[idxt[s]], a[s], sem[s])

        def finish(i, s):
            pltpu.make_async_copy(tok_hbm.at[idxt[s]], a[s], sem[s]).wait()

            @pl.when(i >= NBUF)
            def _():
                pltpu.make_async_copy(
                    c[s], out_hbm.at[page_base + i], wsem[s]).wait()

            @pl.loop(0, IW // 16, unroll=2)
            def add_chunk(ch):
                pv = idxp[s][pl.ds(ch * 16, 16)]
                for k in range(16):
                    r = ch * 16 + k
                    p = pv[k]
                    half = (p & 1) * EMBED_DIM
                    for col in range(EMBED_DIM // 16):
                        sl = (r, pl.ds(col * 16, 16))
                        c[s][sl] = a[s][sl] + pev[
                            p >> 1, pl.ds(half + col * 16, 16)]

            pltpu.async_copy(c[s], out_hbm.at[page_base + i], wsem[s])

        issue(0, 0)

        def pair(m, carry):
            issue(2 * m + 1, 1)
            finish(2 * m, 0)

            @pl.when(2 * m + 2 < STEPS)
            def _():
                issue(2 * m + 2, 0)

            finish(2 * m + 1, 1)
            return carry

        lax.fori_loop(0, STEPS // 2, pair, 0)

        for s in range(NBUF):
            pltpu.make_async_copy(
                c[s], out_hbm.at[page_base + STEPS - NBUF + s], wsem[s]).wait()

    return emb_kernel


_emb = _make_kernel()


@jax.jit
def kernel(x, pos, token_table, position_table):
    tok_p = jnp.pad(token_table, ((0, 0), (0, PADDED - EMBED_DIM)))
    out = _emb(x.reshape(PAGES, IW), pos.reshape(PAGES, IW), tok_p,
               position_table.reshape(MAX_SEQ // 2, 2 * EMBED_DIM))
    return out.reshape(B, L, EMBED_DIM)
